# baseline (speedup 1.0000x reference)
import functools

import jax
import jax.numpy as jnp
from jax import lax
from jax.experimental import pallas as pl
from jax.experimental.pallas import tpu as pltpu
from jax.experimental.pallas import tpu_sc as plsc

N = 10000
E = 320000
D = 128
DO = 48
DEG_W = 16
DX = D + DEG_W

NC, NS = 2, 16
NW = NC * NS
N_PAD = 10240
RPT = N_PAD // NS
CH = 128
N_CHUNKS = E // CH
TRIP = 80
PAIRS = TRIP // 2
BUF = TRIP + 2
EROWS = 2688

_mesh = plsc.VectorSubcoreMesh(
    core_axis_name="c", subcore_axis_name="s", num_cores=NC, num_subcores=NS
)


def _make_agg(d):
  @functools.partial(
      pl.kernel,
      out_type=jax.ShapeDtypeStruct((NC, N_PAD, d), jnp.float32),
      mesh=_mesh,
      scratch_types=[
          pltpu.VMEM((TRIP, CH), jnp.int32),
          pltpu.VMEM((CH,), jnp.int32),
          pltpu.VMEM((CH,), jnp.int32),
          pltpu.VMEM((CH, d), jnp.float32),
          pltpu.VMEM((CH, d), jnp.float32),
          pltpu.VMEM_SHARED((N_PAD, d), jnp.float32),
          pltpu.SemaphoreType.DMA,
          pltpu.SemaphoreType.DMA,
          pltpu.SemaphoreType.DMA,
          pltpu.SemaphoreType.DMA,
          pltpu.SemaphoreType.DMA,
          pltpu.SemaphoreType.DMA,
      ],
      name=f"sage_agg{d}",
      compiler_params=pltpu.CompilerParams(use_tc_tiling_on_sc=False),
  )
  def agg(g_hbm, src_hbm, dst_hbm, zero_hbm, out_hbm,
          dst_all, src_a, src_b, rows_a, rows_b, acc_sh,
          gsa, gsb, ssa, ssb, sca, scb):
    c = lax.axis_index("c")
    s = lax.axis_index("s")
    w = s * NC + c
    r0 = s * RPT
    row0 = w * TRIP
    pltpu.sync_copy(zero_hbm.at[pl.ds(r0, RPT)], acc_sh.at[pl.ds(r0, RPT)])
    pltpu.sync_copy(dst_hbm.at[pl.ds(row0, TRIP)], dst_all)
    plsc.subcore_barrier()

    def src_copy(k, buf, sem):
      return pltpu.async_copy(src_hbm.at[row0 + k], buf, sem)

    def gather(buf, rows, sem):
      return pltpu.async_copy(g_hbm.at[buf], rows, sem)

    def scat(rows, k):
      pltpu.sync_copy(rows, acc_sh.at[dst_all.at[k]], add=True)

    src_copy(0, src_a, ssa).wait()
    src_copy(1, src_b, ssb)
    gather(src_a, rows_a, gsa)

    def body(j, carry):
      k0 = 2 * j
      pltpu.make_async_copy(src_hbm.at[row0], src_b, ssb).wait()
      gather(src_b, rows_b, gsb)
      pltpu.make_async_copy(g_hbm.at[src_a], rows_a, gsa).wait()
      src_copy(k0 + 2, src_a, ssa)
      scat(rows_a, k0)
      pltpu.make_async_copy(src_hbm.at[row0], src_a, ssa).wait()
      gather(src_a, rows_a, gsa)
      pltpu.make_async_copy(g_hbm.at[src_b], rows_b, gsb).wait()
      src_copy(k0 + 3, src_b, ssb)
      scat(rows_b, k0 + 1)
      return carry

    lax.fori_loop(0, PAIRS, body, 0)
    pltpu.make_async_copy(g_hbm.at[src_a], rows_a, gsa).wait()
    pltpu.make_async_copy(src_hbm.at[row0], src_b, ssb).wait()
    plsc.subcore_barrier()
    pltpu.sync_copy(acc_sh.at[pl.ds(r0, RPT)], out_hbm.at[c, pl.ds(r0, RPT)])

  return agg


@functools.partial(
    pl.kernel,
    out_type=jax.ShapeDtypeStruct((NC, N_PAD, DX), jnp.float32),
    mesh=_mesh,
    scratch_types=[
        pltpu.VMEM((CH,), jnp.int32),
        pltpu.VMEM((CH,), jnp.int32),
        pltpu.VMEM((CH,), jnp.int32),
        pltpu.VMEM((CH,), jnp.int32),
        pltpu.VMEM((CH, DX), jnp.float32),
        pltpu.VMEM((CH, DX), jnp.float32),
        pltpu.VMEM_SHARED((N_PAD, DX), jnp.float32),
        pltpu.SemaphoreType.DMA,
        pltpu.SemaphoreType.DMA,
        pltpu.SemaphoreType.DMA,
        pltpu.SemaphoreType.DMA,
        pltpu.SemaphoreType.DMA,
        pltpu.SemaphoreType.DMA,
    ],
    name="sage_agg144",
    compiler_params=pltpu.CompilerParams(use_tc_tiling_on_sc=False),
)
def _agg144(g_hbm, src_hbm, dst_hbm, zero_hbm, out_hbm,
            src_a, src_b, dst_a, dst_b, rows_a, rows_b, acc_sh,
            gsa, gsb, ssa, ssb, dsa, dsb):
  c = lax.axis_index("c")
  s = lax.axis_index("s")
  w = s * NC + c
  r0 = s * RPT
  row0 = w * TRIP
  pltpu.sync_copy(zero_hbm.at[pl.ds(r0, RPT)], acc_sh.at[pl.ds(r0, RPT)])
  plsc.subcore_barrier()

  def idx_copy(hbm, k, buf, sem):
    return pltpu.async_copy(hbm.at[row0 + k], buf, sem)

  def gather(buf, rows, sem):
    return pltpu.async_copy(g_hbm.at[buf], rows, sem)

  def scat(rows, dbuf):
    pltpu.sync_copy(rows, acc_sh.at[dbuf], add=True)

  idx_copy(src_hbm, 0, src_a, ssa).wait()
  idx_copy(src_hbm, 1, src_b, ssb)
  idx_copy(dst_hbm, 0, dst_a, dsa)
  idx_copy(dst_hbm, 1, dst_b, dsb)
  gather(src_a, rows_a, gsa)

  def body(j, carry):
    k0 = 2 * j
    pltpu.make_async_copy(src_hbm.at[row0], src_b, ssb).wait()
    gather(src_b, rows_b, gsb)
    pltpu.make_async_copy(g_hbm.at[src_a], rows_a, gsa).wait()
    idx_copy(src_hbm, k0 + 2, src_a, ssa)
    pltpu.make_async_copy(dst_hbm.at[row0], dst_a, dsa).wait()
    scat(rows_a, dst_a)
    idx_copy(dst_hbm, k0 + 2, dst_a, dsa)
    pltpu.make_async_copy(src_hbm.at[row0], src_a, ssa).wait()
    gather(src_a, rows_a, gsa)
    pltpu.make_async_copy(g_hbm.at[src_b], rows_b, gsb).wait()
    idx_copy(src_hbm, k0 + 3, src_b, ssb)
    pltpu.make_async_copy(dst_hbm.at[row0], dst_b, dsb).wait()
    scat(rows_b, dst_b)
    idx_copy(dst_hbm, k0 + 3, dst_b, dsb)
    return carry

  lax.fori_loop(0, PAIRS, body, 0)
  pltpu.make_async_copy(g_hbm.at[src_a], rows_a, gsa).wait()
  pltpu.make_async_copy(src_hbm.at[row0], src_b, ssb).wait()
  pltpu.make_async_copy(dst_hbm.at[row0], dst_a, dsa).wait()
  pltpu.make_async_copy(dst_hbm.at[row0], dst_b, dsb).wait()
  plsc.subcore_barrier()
  pltpu.sync_copy(acc_sh.at[pl.ds(r0, RPT)], out_hbm.at[c, pl.ds(r0, RPT)])


_agg128 = _make_agg(D)
_agg48 = _make_agg(DO)

R = 2000
GRID = N // R


def _mm_in_body(x_ref, ws_ref, wn_ref, b_ref, s_ref, g_ref):
  x = x_ref[...]
  s_ref[...] = jnp.dot(x, ws_ref[...], preferred_element_type=jnp.float32) + b_ref[...]
  g_ref[:, :D] = jnp.dot(x, wn_ref[...], preferred_element_type=jnp.float32)
  g_ref[:, D:] = jnp.ones((R, DEG_W), jnp.float32)


def _comb1_body(s_ref, p_ref, ws_ref, wn_ref, b_ref, so_ref, go_ref):
  p = p_ref[0, :, :D] + p_ref[1, :, :D]
  deg = p_ref[0, :, D:D + 1] + p_ref[1, :, D:D + 1]
  inv = 1.0 / jnp.maximum(deg, 1.0)
  h = jnp.maximum(s_ref[...] + p * inv, 0.0)
  so_ref[...] = jnp.dot(h, ws_ref[...], preferred_element_type=jnp.float32) + b_ref[...]
  go_ref[...] = jnp.dot(h, wn_ref[...], preferred_element_type=jnp.float32)


def _comb_body(s_ref, p_ref, d_ref, ws_ref, wn_ref, b_ref, so_ref, go_ref):
  p = p_ref[0] + p_ref[1]
  deg = d_ref[0, :, D:D + 1] + d_ref[1, :, D:D + 1]
  inv = 1.0 / jnp.maximum(deg, 1.0)
  h = jnp.maximum(s_ref[...] + p * inv, 0.0)
  so_ref[...] = jnp.dot(h, ws_ref[...], preferred_element_type=jnp.float32) + b_ref[...]
  go_ref[...] = jnp.dot(h, wn_ref[...], preferred_element_type=jnp.float32)


def _final_body(s_ref, p_ref, d_ref, o_ref):
  p = p_ref[0] + p_ref[1]
  deg = d_ref[0, :, D:D + 1] + d_ref[1, :, D:D + 1]
  inv = 1.0 / jnp.maximum(deg, 1.0)
  z = s_ref[...] + p * inv
  valid = lax.broadcasted_iota(jnp.int32, z.shape, 1) < 40
  zm = jnp.where(valid, z, -jnp.inf)
  m = jnp.max(zm, axis=1, keepdims=True)
  e = jnp.where(valid, jnp.exp(z - m), 0.0)
  lse = jnp.log(jnp.sum(e, axis=1, keepdims=True)) + m
  o_ref[...] = (z - lse)[:, :40]


def _row_spec(d):
  return pl.BlockSpec((R, d), lambda i: (i, 0))


def _part_spec(d):
  return pl.BlockSpec((NC, R, d), lambda i: (0, i, 0))


def _full_spec(a, b):
  return pl.BlockSpec((a, b), lambda i: (0, 0))


_deg_spec = _part_spec(DX)

_mm_in = pl.pallas_call(
    _mm_in_body,
    grid=(GRID,),
    in_specs=[_row_spec(D), _full_spec(D, D), _full_spec(D, D), _full_spec(1, D)],
    out_specs=[_row_spec(D), _row_spec(DX)],
    out_shape=[jax.ShapeDtypeStruct((N, D), jnp.float32),
               jax.ShapeDtypeStruct((N, DX), jnp.float32)],
)

_comb1 = pl.pallas_call(
    _comb1_body,
    grid=(GRID,),
    in_specs=[_row_spec(D), _part_spec(DX),
              _full_spec(D, D), _full_spec(D, D), _full_spec(1, D)],
    out_specs=[_row_spec(D), _row_spec(D)],
    out_shape=[jax.ShapeDtypeStruct((N, D), jnp.float32)] * 2,
)

_comb2 = pl.pallas_call(
    _comb_body,
    grid=(GRID,),
    in_specs=[_row_spec(D), _part_spec(D), _deg_spec,
              _full_spec(D, DO), _full_spec(D, DO), _full_spec(1, DO)],
    out_specs=[_row_spec(DO), _row_spec(DO)],
    out_shape=[jax.ShapeDtypeStruct((N, DO), jnp.float32)] * 2,
)

_final = pl.pallas_call(
    _final_body,
    grid=(GRID,),
    in_specs=[_row_spec(DO), _part_spec(DO), _deg_spec],
    out_specs=pl.BlockSpec((R, 40), lambda i: (i, 0)),
    out_shape=jax.ShapeDtypeStruct((N, 40), jnp.float32),
)


def kernel(x, edge_index, W_self0, W_neigh0, b0,
           W_self1, W_neigh1, b1, W_self2, W_neigh2, b2):
  src = edge_index[0].astype(jnp.int32)
  dst = edge_index[1].astype(jnp.int32)
  n_dummy = EROWS * CH - E
  dummy_src = jnp.arange(n_dummy, dtype=jnp.int32) % N
  dummy_dst = jnp.arange(n_dummy, dtype=jnp.int32) % (N_PAD - N) + N
  src2 = jnp.concatenate([src, dummy_src]).reshape(EROWS, CH)
  dst2 = jnp.concatenate([dst, dummy_dst]).reshape(EROWS, CH)
  zeros128 = jnp.zeros((N_PAD, D), jnp.float32)
  zeros48 = jnp.zeros((N_PAD, DO), jnp.float32)
  zeros144 = jnp.zeros((N_PAD, DX), jnp.float32)

  s0, g0 = _mm_in(x, W_self0, W_neigh0, b0[None])
  p0 = _agg144(g0, src2, dst2, zeros144)
  s1, g1 = _comb1(s0, p0, W_self1, W_neigh1, b1[None])
  p1 = _agg128(g1, src2, dst2, zeros128)
  ws2 = jnp.pad(W_self2, ((0, 0), (0, DO - 40)))
  wn2 = jnp.pad(W_neigh2, ((0, 0), (0, DO - 40)))
  b2p = jnp.pad(b2, (0, DO - 40))
  s2, g2 = _comb2(s1, p1, p0, ws2, wn2, b2p[None])
  p2 = _agg48(g2, src2, dst2, zeros48)
  return _final(s2, p2, p0)


# speedup vs baseline: 1.0870x; 1.0870x over previous
import functools

import jax
import jax.numpy as jnp
from jax import lax
from jax.experimental import pallas as pl
from jax.experimental.pallas import tpu as pltpu
from jax.experimental.pallas import tpu_sc as plsc

N = 10000
E = 320000
D = 128
DO = 48
DEG_W = 16

NC, NS = 2, 16
NW = NC * NS
N_PAD = 10240
RPT = N_PAD // NS
CH = 128
N_CHUNKS = E // CH
TRIP = 80
PAIRS = TRIP // 2
BUF = TRIP + 2
EROWS = 2688

_mesh = plsc.VectorSubcoreMesh(
    core_axis_name="c", subcore_axis_name="s", num_cores=NC, num_subcores=NS
)


def _make_agg(d):
  @functools.partial(
      pl.kernel,
      out_type=jax.ShapeDtypeStruct((NC, N_PAD, d), jnp.float32),
      mesh=_mesh,
      scratch_types=[
          pltpu.VMEM((TRIP, CH), jnp.int32),
          pltpu.VMEM((CH,), jnp.int32),
          pltpu.VMEM((CH,), jnp.int32),
          pltpu.VMEM((CH, d), jnp.float32),
          pltpu.VMEM((CH, d), jnp.float32),
          pltpu.VMEM_SHARED((N_PAD, d), jnp.float32),
          pltpu.SemaphoreType.DMA,
          pltpu.SemaphoreType.DMA,
          pltpu.SemaphoreType.DMA,
          pltpu.SemaphoreType.DMA,
          pltpu.SemaphoreType.DMA,
          pltpu.SemaphoreType.DMA,
      ],
      name=f"sage_agg{d}",
      compiler_params=pltpu.CompilerParams(use_tc_tiling_on_sc=False),
  )
  def agg(g_hbm, src_hbm, dst_hbm, zero_hbm, out_hbm,
          dst_all, src_a, src_b, rows_a, rows_b, acc_sh,
          gsa, gsb, ssa, ssb, sca, scb):
    c = lax.axis_index("c")
    s = lax.axis_index("s")
    w = s * NC + c
    r0 = s * RPT
    row0 = w * TRIP
    pltpu.sync_copy(zero_hbm.at[pl.ds(r0, RPT)], acc_sh.at[pl.ds(r0, RPT)])
    pltpu.sync_copy(dst_hbm.at[pl.ds(row0, TRIP)], dst_all)
    plsc.subcore_barrier()

    def src_copy(k, buf, sem):
      return pltpu.async_copy(src_hbm.at[row0 + k], buf, sem)

    def gather(buf, rows, sem):
      return pltpu.async_copy(g_hbm.at[buf], rows, sem)

    def scat(rows, k):
      pltpu.sync_copy(rows, acc_sh.at[dst_all.at[k]], add=True)

    src_copy(0, src_a, ssa).wait()
    src_copy(1, src_b, ssb)
    gather(src_a, rows_a, gsa)

    def body(j, carry):
      k0 = 2 * j
      pltpu.make_async_copy(src_hbm.at[row0], src_b, ssb).wait()
      gather(src_b, rows_b, gsb)
      pltpu.make_async_copy(g_hbm.at[src_a], rows_a, gsa).wait()
      src_copy(k0 + 2, src_a, ssa)
      scat(rows_a, k0)
      pltpu.make_async_copy(src_hbm.at[row0], src_a, ssa).wait()
      gather(src_a, rows_a, gsa)
      pltpu.make_async_copy(g_hbm.at[src_b], rows_b, gsb).wait()
      src_copy(k0 + 3, src_b, ssb)
      scat(rows_b, k0 + 1)
      return carry

    lax.fori_loop(0, PAIRS, body, 0)
    pltpu.make_async_copy(g_hbm.at[src_a], rows_a, gsa).wait()
    pltpu.make_async_copy(src_hbm.at[row0], src_b, ssb).wait()
    plsc.subcore_barrier()
    pltpu.sync_copy(acc_sh.at[pl.ds(r0, RPT)], out_hbm.at[c, pl.ds(r0, RPT)])

  return agg


@functools.partial(
    pl.kernel,
    out_type=jax.ShapeDtypeStruct((NC, N_PAD, DEG_W), jnp.float32),
    mesh=_mesh,
    scratch_types=[
        pltpu.VMEM((TRIP, CH), jnp.int32),
        pltpu.VMEM((CH, DEG_W), jnp.float32),
        pltpu.VMEM_SHARED((N_PAD, DEG_W), jnp.float32),
    ],
    name="sage_deg",
    compiler_params=pltpu.CompilerParams(use_tc_tiling_on_sc=False),
)
def _deg(dst_hbm, zero_hbm, ones_hbm, out_hbm, dst_all, ones_v, acc_sh):
  c = lax.axis_index("c")
  s = lax.axis_index("s")
  w = s * NC + c
  r0 = s * RPT
  pltpu.sync_copy(zero_hbm.at[pl.ds(r0, RPT)], acc_sh.at[pl.ds(r0, RPT)])
  pltpu.sync_copy(ones_hbm, ones_v)
  pltpu.sync_copy(dst_hbm.at[pl.ds(w * TRIP, TRIP)], dst_all)
  plsc.subcore_barrier()

  def body(k, carry):
    pltpu.sync_copy(ones_v, acc_sh.at[dst_all.at[k]], add=True)
    return carry

  lax.fori_loop(0, TRIP, body, 0)
  plsc.subcore_barrier()
  pltpu.sync_copy(acc_sh.at[pl.ds(r0, RPT)], out_hbm.at[c, pl.ds(r0, RPT)])


_agg128 = _make_agg(D)
_agg48 = _make_agg(DO)

R = 2000
GRID = N // R


def _mm_in_body(x_ref, ws_ref, wn_ref, b_ref, s_ref, g_ref):
  x = x_ref[...]
  s_ref[...] = jnp.dot(x, ws_ref[...], preferred_element_type=jnp.float32) + b_ref[...]
  g_ref[...] = jnp.dot(x, wn_ref[...], preferred_element_type=jnp.float32)


def _comb_body(s_ref, p_ref, d_ref, ws_ref, wn_ref, b_ref, so_ref, go_ref):
  p = p_ref[0] + p_ref[1]
  deg = d_ref[0, :, :1] + d_ref[1, :, :1]
  inv = 1.0 / jnp.maximum(deg, 1.0)
  h = jnp.maximum(s_ref[...] + p * inv, 0.0)
  so_ref[...] = jnp.dot(h, ws_ref[...], preferred_element_type=jnp.float32) + b_ref[...]
  go_ref[...] = jnp.dot(h, wn_ref[...], preferred_element_type=jnp.float32)


def _final_body(s_ref, p_ref, d_ref, o_ref):
  p = p_ref[0] + p_ref[1]
  deg = d_ref[0, :, :1] + d_ref[1, :, :1]
  inv = 1.0 / jnp.maximum(deg, 1.0)
  z = s_ref[...] + p * inv
  valid = lax.broadcasted_iota(jnp.int32, z.shape, 1) < 40
  zm = jnp.where(valid, z, -jnp.inf)
  m = jnp.max(zm, axis=1, keepdims=True)
  e = jnp.where(valid, jnp.exp(z - m), 0.0)
  lse = jnp.log(jnp.sum(e, axis=1, keepdims=True)) + m
  o_ref[...] = (z - lse)[:, :40]


def _row_spec(d):
  return pl.BlockSpec((R, d), lambda i: (i, 0))


def _part_spec(d):
  return pl.BlockSpec((NC, R, d), lambda i: (0, i, 0))


def _full_spec(a, b):
  return pl.BlockSpec((a, b), lambda i: (0, 0))


_mm_in = pl.pallas_call(
    _mm_in_body,
    grid=(GRID,),
    in_specs=[_row_spec(D), _full_spec(D, D), _full_spec(D, D), _full_spec(1, D)],
    out_specs=[_row_spec(D), _row_spec(D)],
    out_shape=[jax.ShapeDtypeStruct((N, D), jnp.float32)] * 2,
)

_comb1 = pl.pallas_call(
    _comb_body,
    grid=(GRID,),
    in_specs=[_row_spec(D), _part_spec(D), _part_spec(DEG_W),
              _full_spec(D, D), _full_spec(D, D), _full_spec(1, D)],
    out_specs=[_row_spec(D), _row_spec(D)],
    out_shape=[jax.ShapeDtypeStruct((N, D), jnp.float32)] * 2,
)

_comb2 = pl.pallas_call(
    _comb_body,
    grid=(GRID,),
    in_specs=[_row_spec(D), _part_spec(D), _part_spec(DEG_W),
              _full_spec(D, DO), _full_spec(D, DO), _full_spec(1, DO)],
    out_specs=[_row_spec(DO), _row_spec(DO)],
    out_shape=[jax.ShapeDtypeStruct((N, DO), jnp.float32)] * 2,
)

_final = pl.pallas_call(
    _final_body,
    grid=(GRID,),
    in_specs=[_row_spec(DO), _part_spec(DO), _part_spec(DEG_W)],
    out_specs=pl.BlockSpec((R, 40), lambda i: (i, 0)),
    out_shape=jax.ShapeDtypeStruct((N, 40), jnp.float32),
)


def kernel(x, edge_index, W_self0, W_neigh0, b0,
           W_self1, W_neigh1, b1, W_self2, W_neigh2, b2):
  src = edge_index[0].astype(jnp.int32)
  dst = edge_index[1].astype(jnp.int32)
  n_dummy = EROWS * CH - E
  dummy_src = jnp.arange(n_dummy, dtype=jnp.int32) % N
  dummy_dst = jnp.arange(n_dummy, dtype=jnp.int32) % (N_PAD - N) + N
  src2 = jnp.concatenate([src, dummy_src]).reshape(EROWS, CH)
  dst2 = jnp.concatenate([dst, dummy_dst]).reshape(EROWS, CH)
  zeros128 = jnp.zeros((N_PAD, D), jnp.float32)
  zeros48 = jnp.zeros((N_PAD, DO), jnp.float32)
  zeros16 = jnp.zeros((N_PAD, DEG_W), jnp.float32)
  ones16 = jnp.ones((CH, DEG_W), jnp.float32)

  deg = _deg(dst2, zeros16, ones16)
  s0, g0 = _mm_in(x, W_self0, W_neigh0, b0[None])
  p0 = _agg128(g0, src2, dst2, zeros128)
  s1, g1 = _comb1(s0, p0, deg, W_self1, W_neigh1, b1[None])
  p1 = _agg128(g1, src2, dst2, zeros128)
  ws2 = jnp.pad(W_self2, ((0, 0), (0, DO - 40)))
  wn2 = jnp.pad(W_neigh2, ((0, 0), (0, DO - 40)))
  b2p = jnp.pad(b2, (0, DO - 40))
  s2, g2 = _comb2(s1, p1, deg, ws2, wn2, b2p[None])
  p2 = _agg48(g2, src2, dst2, zeros48)
  return _final(s2, p2, deg)


# speedup vs baseline: 1.1055x; 1.0171x over previous
import functools

import jax
import jax.numpy as jnp
from jax import lax
from jax.experimental import pallas as pl
from jax.experimental.pallas import tpu as pltpu
from jax.experimental.pallas import tpu_sc as plsc

N = 10000
E = 320000
D = 128
DO = 48
DEG_W = 16

NC, NS = 2, 16
NW = NC * NS
N_PAD = 10240
RPT = N_PAD // NS
CH = 128
N_CHUNKS = E // CH
TRIP = 80
PAIRS = TRIP // 2
BUF = TRIP + 2
EROWS = 2688

_mesh = plsc.VectorSubcoreMesh(
    core_axis_name="c", subcore_axis_name="s", num_cores=NC, num_subcores=NS
)


def _make_agg(d, preload_src=False):
  src_shape = (BUF, CH) if preload_src else (CH,)

  @functools.partial(
      pl.kernel,
      out_type=jax.ShapeDtypeStruct((NC, N_PAD, d), jnp.float32),
      mesh=_mesh,
      scratch_types=[
          pltpu.VMEM((TRIP, CH), jnp.int32),
          pltpu.VMEM(src_shape, jnp.int32),
          pltpu.VMEM((CH,), jnp.int32),
          pltpu.VMEM((CH, d), jnp.float32),
          pltpu.VMEM((CH, d), jnp.float32),
          pltpu.VMEM_SHARED((N_PAD, d), jnp.float32),
          pltpu.SemaphoreType.DMA,
          pltpu.SemaphoreType.DMA,
          pltpu.SemaphoreType.DMA,
          pltpu.SemaphoreType.DMA,
          pltpu.SemaphoreType.DMA,
          pltpu.SemaphoreType.DMA,
      ],
      name=f"sage_agg{d}",
      compiler_params=pltpu.CompilerParams(use_tc_tiling_on_sc=False),
  )
  def agg(g_hbm, src_hbm, dst_hbm, zero_hbm, out_hbm,
          dst_all, src_a, src_b, rows_a, rows_b, acc_sh,
          gsa, gsb, ssa, ssb, sca, scb):
    c = lax.axis_index("c")
    s = lax.axis_index("s")
    w = s * NC + c
    r0 = s * RPT
    row0 = w * TRIP
    pltpu.sync_copy(zero_hbm.at[pl.ds(r0, RPT)], acc_sh.at[pl.ds(r0, RPT)])
    pltpu.sync_copy(dst_hbm.at[pl.ds(row0, TRIP)], dst_all)
    if preload_src:
      pltpu.sync_copy(src_hbm.at[pl.ds(row0, BUF)], src_a)
    plsc.subcore_barrier()

    def src_copy(k, buf, sem):
      return pltpu.async_copy(src_hbm.at[row0 + k], buf, sem)

    def gather(buf, rows, sem):
      return pltpu.async_copy(g_hbm.at[buf], rows, sem)

    def scat(rows, k):
      pltpu.sync_copy(rows, acc_sh.at[dst_all.at[k]], add=True)

    if preload_src:
      gather(src_a.at[0], rows_a, gsa)

      def body(j, carry):
        k0 = 2 * j
        gather(src_a.at[k0 + 1], rows_b, gsb)
        pltpu.make_async_copy(g_hbm.at[src_b], rows_a, gsa).wait()
        scat(rows_a, k0)
        gather(src_a.at[k0 + 2], rows_a, gsa)
        pltpu.make_async_copy(g_hbm.at[src_b], rows_b, gsb).wait()
        scat(rows_b, k0 + 1)
        return carry

      lax.fori_loop(0, PAIRS, body, 0)
      pltpu.make_async_copy(g_hbm.at[src_b], rows_a, gsa).wait()
    else:
      src_copy(0, src_a, ssa).wait()
      src_copy(1, src_b, ssb)
      gather(src_a, rows_a, gsa)

      def body(j, carry):
        k0 = 2 * j
        pltpu.make_async_copy(src_hbm.at[row0], src_b, ssb).wait()
        gather(src_b, rows_b, gsb)
        pltpu.make_async_copy(g_hbm.at[src_a], rows_a, gsa).wait()
        src_copy(k0 + 2, src_a, ssa)
        scat(rows_a, k0)
        pltpu.make_async_copy(src_hbm.at[row0], src_a, ssa).wait()
        gather(src_a, rows_a, gsa)
        pltpu.make_async_copy(g_hbm.at[src_b], rows_b, gsb).wait()
        src_copy(k0 + 3, src_b, ssb)
        scat(rows_b, k0 + 1)
        return carry

      lax.fori_loop(0, PAIRS, body, 0)
      pltpu.make_async_copy(g_hbm.at[src_a], rows_a, gsa).wait()
      pltpu.make_async_copy(src_hbm.at[row0], src_b, ssb).wait()
    plsc.subcore_barrier()
    pltpu.sync_copy(acc_sh.at[pl.ds(r0, RPT)], out_hbm.at[c, pl.ds(r0, RPT)])

  return agg


@functools.partial(
    pl.kernel,
    out_type=jax.ShapeDtypeStruct((NC, N_PAD, DEG_W), jnp.float32),
    mesh=_mesh,
    scratch_types=[
        pltpu.VMEM((TRIP, CH), jnp.int32),
        pltpu.VMEM((CH, DEG_W), jnp.float32),
        pltpu.VMEM_SHARED((N_PAD, DEG_W), jnp.float32),
    ],
    name="sage_deg",
    compiler_params=pltpu.CompilerParams(use_tc_tiling_on_sc=False),
)
def _deg(dst_hbm, zero_hbm, ones_hbm, out_hbm, dst_all, ones_v, acc_sh):
  c = lax.axis_index("c")
  s = lax.axis_index("s")
  w = s * NC + c
  r0 = s * RPT
  pltpu.sync_copy(zero_hbm.at[pl.ds(r0, RPT)], acc_sh.at[pl.ds(r0, RPT)])
  pltpu.sync_copy(ones_hbm, ones_v)
  pltpu.sync_copy(dst_hbm.at[pl.ds(w * TRIP, TRIP)], dst_all)
  plsc.subcore_barrier()

  def body(k, carry):
    pltpu.sync_copy(ones_v, acc_sh.at[dst_all.at[k]], add=True)
    return carry

  lax.fori_loop(0, TRIP, body, 0)
  plsc.subcore_barrier()
  pltpu.sync_copy(acc_sh.at[pl.ds(r0, RPT)], out_hbm.at[c, pl.ds(r0, RPT)])


_agg128 = _make_agg(D)
_agg48 = _make_agg(DO, preload_src=True)

R = 2000
GRID = N // R


def _mm_in_body(x_ref, ws_ref, wn_ref, b_ref, s_ref, g_ref):
  x = x_ref[...]
  s_ref[...] = jnp.dot(x, ws_ref[...], preferred_element_type=jnp.float32) + b_ref[...]
  g_ref[...] = jnp.dot(x, wn_ref[...], preferred_element_type=jnp.float32)


def _comb_body(s_ref, p_ref, d_ref, ws_ref, wn_ref, b_ref, so_ref, go_ref):
  p = p_ref[0] + p_ref[1]
  deg = d_ref[0, :, :1] + d_ref[1, :, :1]
  inv = 1.0 / jnp.maximum(deg, 1.0)
  h = jnp.maximum(s_ref[...] + p * inv, 0.0)
  so_ref[...] = jnp.dot(h, ws_ref[...], preferred_element_type=jnp.float32) + b_ref[...]
  go_ref[...] = jnp.dot(h, wn_ref[...], preferred_element_type=jnp.float32)


def _final_body(s_ref, p_ref, d_ref, o_ref):
  p = p_ref[0] + p_ref[1]
  deg = d_ref[0, :, :1] + d_ref[1, :, :1]
  inv = 1.0 / jnp.maximum(deg, 1.0)
  z = s_ref[...] + p * inv
  valid = lax.broadcasted_iota(jnp.int32, z.shape, 1) < 40
  zm = jnp.where(valid, z, -jnp.inf)
  m = jnp.max(zm, axis=1, keepdims=True)
  e = jnp.where(valid, jnp.exp(z - m), 0.0)
  lse = jnp.log(jnp.sum(e, axis=1, keepdims=True)) + m
  o_ref[...] = (z - lse)[:, :40]


def _row_spec(d):
  return pl.BlockSpec((R, d), lambda i: (i, 0))


def _part_spec(d):
  return pl.BlockSpec((NC, R, d), lambda i: (0, i, 0))


def _full_spec(a, b):
  return pl.BlockSpec((a, b), lambda i: (0, 0))


_mm_in = pl.pallas_call(
    _mm_in_body,
    grid=(GRID,),
    in_specs=[_row_spec(D), _full_spec(D, D), _full_spec(D, D), _full_spec(1, D)],
    out_specs=[_row_spec(D), _row_spec(D)],
    out_shape=[jax.ShapeDtypeStruct((N, D), jnp.float32)] * 2,
)

_comb1 = pl.pallas_call(
    _comb_body,
    grid=(GRID,),
    in_specs=[_row_spec(D), _part_spec(D), _part_spec(DEG_W),
              _full_spec(D, D), _full_spec(D, D), _full_spec(1, D)],
    out_specs=[_row_spec(D), _row_spec(D)],
    out_shape=[jax.ShapeDtypeStruct((N, D), jnp.float32)] * 2,
)

_comb2 = pl.pallas_call(
    _comb_body,
    grid=(GRID,),
    in_specs=[_row_spec(D), _part_spec(D), _part_spec(DEG_W),
              _full_spec(D, DO), _full_spec(D, DO), _full_spec(1, DO)],
    out_specs=[_row_spec(DO), _row_spec(DO)],
    out_shape=[jax.ShapeDtypeStruct((N, DO), jnp.float32)] * 2,
)

_final = pl.pallas_call(
    _final_body,
    grid=(GRID,),
    in_specs=[_row_spec(DO), _part_spec(DO), _part_spec(DEG_W)],
    out_specs=pl.BlockSpec((R, 40), lambda i: (i, 0)),
    out_shape=jax.ShapeDtypeStruct((N, 40), jnp.float32),
)


def kernel(x, edge_index, W_self0, W_neigh0, b0,
           W_self1, W_neigh1, b1, W_self2, W_neigh2, b2):
  src = edge_index[0].astype(jnp.int32)
  dst = edge_index[1].astype(jnp.int32)
  n_dummy = EROWS * CH - E
  dummy_src = jnp.arange(n_dummy, dtype=jnp.int32) % N
  dummy_dst = jnp.arange(n_dummy, dtype=jnp.int32) % (N_PAD - N) + N
  src2 = jnp.concatenate([src, dummy_src]).reshape(EROWS, CH)
  dst2 = jnp.concatenate([dst, dummy_dst]).reshape(EROWS, CH)
  zeros128 = jnp.zeros((N_PAD, D), jnp.float32)
  zeros48 = jnp.zeros((N_PAD, DO), jnp.float32)
  zeros16 = jnp.zeros((N_PAD, DEG_W), jnp.float32)
  ones16 = jnp.ones((CH, DEG_W), jnp.float32)

  deg = _deg(dst2, zeros16, ones16)
  s0, g0 = _mm_in(x, W_self0, W_neigh0, b0[None])
  p0 = _agg128(g0, src2, dst2, zeros128)
  s1, g1 = _comb1(s0, p0, deg, W_self1, W_neigh1, b1[None])
  p1 = _agg128(g1, src2, dst2, zeros128)
  ws2 = jnp.pad(W_self2, ((0, 0), (0, DO - 40)))
  wn2 = jnp.pad(W_neigh2, ((0, 0), (0, DO - 40)))
  b2p = jnp.pad(b2, (0, DO - 40))
  s2, g2 = _comb2(s1, p1, deg, ws2, wn2, b2p[None])
  p2 = _agg48(g2, src2, dst2, zeros48)
  return _final(s2, p2, deg)
